# baseline (speedup 1.0000x reference)
import functools

import jax
import jax.numpy as jnp
from jax import lax
from jax.experimental import pallas as pl
from jax.experimental.pallas import tpu as pltpu
from jax.experimental.pallas import tpu_sc as plsc

_NUM_CORES = 2
_NUM_SUBCORES = 16
_NUM_WORKERS = _NUM_CORES * _NUM_SUBCORES


def kernel(x, table):
    (batch,) = x.shape
    _, dim = table.shape
    b_per_w = batch // _NUM_WORKERS

    idx = x.astype(jnp.int32)
    mesh = plsc.VectorSubcoreMesh(
        core_axis_name="c", subcore_axis_name="s",
        num_cores=_NUM_CORES, num_subcores=_NUM_SUBCORES)

    @functools.partial(
        pl.kernel,
        out_type=jax.ShapeDtypeStruct((batch, dim), table.dtype),
        mesh=mesh,
        scratch_types=[
            pltpu.VMEM((b_per_w,), jnp.int32),
            pltpu.VMEM((b_per_w, dim), jnp.float32),
            pltpu.SemaphoreType.DMA,
        ],
    )
    def emb(idx_hbm, table_hbm, out_hbm, idx_v, rows_v, sem):
        wid = lax.axis_index("s") * _NUM_CORES + lax.axis_index("c")
        base = wid * b_per_w
        pltpu.sync_copy(idx_hbm.at[pl.ds(base, b_per_w)], idx_v)

        def body(j, carry):
            for g in range(4):
                v = idx_v[pl.ds(j * 64 + g * 16, 16)]
                for k in range(16):
                    r = v[k]
                    pltpu.make_async_copy(
                        table_hbm.at[pl.ds(r, 1), :],
                        out_hbm.at[pl.ds(base + j * 64 + g * 16 + k, 1), :],
                        sem).start()
            return carry

        lax.fori_loop(0, b_per_w // 64, body, 0)
        pltpu.make_async_copy(
            table_hbm.at[pl.ds(0, b_per_w), :],
            out_hbm.at[pl.ds(base, b_per_w), :],
            sem).wait()

    return emb(idx, table)


# speedup vs baseline: 4.7410x; 4.7410x over previous
import functools

import jax
import jax.numpy as jnp
from jax import lax
from jax.experimental import pallas as pl
from jax.experimental.pallas import tpu as pltpu
from jax.experimental.pallas import tpu_sc as plsc

_NUM_CORES = 2
_NUM_SUBCORES = 16
_NUM_WORKERS = _NUM_CORES * _NUM_SUBCORES


def kernel(x, table):
    (batch,) = x.shape
    _, dim = table.shape
    b_per_w = batch // _NUM_WORKERS

    idx = x.astype(jnp.int32)
    mesh = plsc.VectorSubcoreMesh(
        core_axis_name="c", subcore_axis_name="s",
        num_cores=_NUM_CORES, num_subcores=_NUM_SUBCORES)

    @functools.partial(
        pl.kernel,
        out_type=jax.ShapeDtypeStruct((batch, dim), table.dtype),
        mesh=mesh,
        scratch_types=[
            pltpu.VMEM((b_per_w,), jnp.int32),
            pltpu.VMEM((b_per_w, dim), jnp.float32),
            pltpu.SemaphoreType.DMA,
        ],
    )
    def emb(idx_hbm, table_hbm, out_hbm, idx_v, rows_v, sem):
        wid = lax.axis_index("s") * _NUM_CORES + lax.axis_index("c")
        base = wid * b_per_w
        pltpu.sync_copy(idx_hbm.at[pl.ds(base, b_per_w)], idx_v)

        def body(j, carry):
            for g in range(4):
                v = idx_v[pl.ds(j * 64 + g * 16, 16)]
                for k in range(16):
                    r = v[k]
                    pltpu.make_async_copy(
                        table_hbm.at[pl.ds(r, 1), :],
                        rows_v.at[pl.ds(j * 64 + g * 16 + k, 1), :],
                        sem).start()
            return carry

        lax.fori_loop(0, b_per_w // 64, body, 0)
        pltpu.make_async_copy(
            table_hbm.at[pl.ds(0, b_per_w), :], rows_v, sem).wait()
        pltpu.sync_copy(rows_v, out_hbm.at[pl.ds(base, b_per_w), :])

    return emb(idx, table)
